# baseline (speedup 1.0000x reference)
import jax
import jax.numpy as jnp
from jax.experimental import pallas as pl


_BLOCK_S = 2048


def _add_kernel(x_ref, e_ref, o_ref):
    o_ref[...] = x_ref[...] + e_ref[...]


def kernel(x, encoding):
    N, S, D = x.shape
    x2 = x.reshape(S, D)
    out = pl.pallas_call(
        _add_kernel,
        out_shape=jax.ShapeDtypeStruct((S, D), x.dtype),
        grid=(S // _BLOCK_S,),
        in_specs=[
            pl.BlockSpec((_BLOCK_S, D), lambda i: (i, 0)),
            pl.BlockSpec((_BLOCK_S, D), lambda i: (i, 0)),
        ],
        out_specs=pl.BlockSpec((_BLOCK_S, D), lambda i: (i, 0)),
    )(x2, encoding)
    return out.reshape(N, S, D)
